# Initial kernel scaffold; baseline (speedup 1.0000x reference)
import jax
import jax.numpy as jnp
from jax.experimental import pallas as pl


def kernel(score, nidx, tidx, specweight):
    raise NotImplementedError("write your pallas kernel here")


# speedup vs baseline: 116.9461x; 116.9461x over previous
import functools

import jax
import jax.numpy as jnp
from jax import lax
from jax.experimental import pallas as pl
from jax.experimental.pallas import tpu as pltpu
from jax.experimental.pallas import tpu_sc as plsc

NC = 2
NS = 16
NW = NC * NS
L = 16

V = 100000
KN = 33
KM = 32

VPT = 3136
C = 224
CHUNKS = 14

EPS = 1e-7
LN2 = 0.6931471805599453
P0 = -2.786812953867443
P1 = 5.046876044975941
P2 = -3.49249427987935
P3 = 1.5939013634991297
P4 = -0.4048671744191854
P5 = 0.043428907822139526

_mesh = plsc.VectorSubcoreMesh(core_axis_name="c", subcore_axis_name="s")

_SCRATCH = [
    pltpu.VMEM((V,), jnp.int32),
    pltpu.VMEM((C * KN,), jnp.int32),
    pltpu.VMEM((C * KM,), jnp.float32),
    pltpu.VMEM((C,), jnp.int32),
    pltpu.VMEM((C,), jnp.float32),
    pltpu.VMEM((C * KM,), jnp.float32),
    pltpu.VMEM((L,), jnp.float32),
]


def _edge_loss_body(packed_hbm, nidx_hbm, score_hbm, tidx_hbm, sw_hbm, out_hbm,
                    table_v, nidx_v, score_v, tidx_v, sw_v, term_v, out_v):
    wid = lax.axis_index("s") * NC + lax.axis_index("c")
    pltpu.sync_copy(packed_hbm, table_v)

    lanes = lax.iota(jnp.int32, L)
    lanes_km = lanes * KM

    def chunk_body(c, acc):
        ub = wid * VPT + c * C
        base = jnp.minimum(ub, V - C)
        pltpu.sync_copy(nidx_hbm.at[pl.ds(base * KN, C * KN)], nidx_v)
        pltpu.sync_copy(score_hbm.at[pl.ds(base * KM, C * KM)], score_v)
        pltpu.sync_copy(tidx_hbm.at[pl.ds(base, C)], tidx_v)
        pltpu.sync_copy(sw_hbm.at[pl.ds(base, C)], sw_v)

        @pl.loop(0, C)
        def _pass1(i):
            tv = plsc.load_gather(tidx_v, [jnp.full((L,), i, jnp.int32)])
            for h in range(2):
                nid = nidx_v[pl.ds(i * KN + 1 + h * L, L)]
                msk = nid >= 0
                safe = jnp.maximum(nid, 0)
                g = plsc.load_gather(table_v, [safe])
                nt = g >> 16
                wgt = plsc.bitcast(g << 16, jnp.float32)
                w = jnp.where(msk, wgt, 0.0)
                same = msk & (nt >= 0) & (nt == tv)
                p = score_v[pl.ds(i * KM + h * L, L)]
                p = jnp.minimum(jnp.maximum(p, EPS), 1.0 - EPS)
                q = jnp.where(same, p, 1.0 - p)
                qb = plsc.bitcast(q, jnp.int32)
                e = ((qb >> 23) & 0xFF) - 127
                m = plsc.bitcast((qb & 0x007FFFFF) | 0x3F800000, jnp.float32)
                poly = P5
                for coef in (P4, P3, P2, P1, P0):
                    poly = poly * m + coef
                logq = (e.astype(jnp.float32) + poly) * LN2
                term_v[pl.ds(i * KM + h * L, L)] = (-logq) * w

        def group_body(vb, acc2):
            rowbase = vb * L
            num = jnp.zeros((L,), jnp.float32)
            den = jnp.zeros((L,), jnp.float32)
            for k in range(KM):
                idx = lanes_km + (rowbase * KM + k)
                t = plsc.load_gather(term_v, [idx])
                num = num + t
                den = den + jnp.where(t > 0.0, 1.0, 0.0)
            swv = sw_v[pl.ds(rowbase, L)]
            swv = jnp.minimum(jnp.maximum(swv, 0.0), 1.0)
            safe_den = jnp.where(den > 0.0, den, 1.0)
            pv = jnp.where(den > 0.0, num / safe_den, 0.0) * (1.0 - 0.9 * swv)
            gvid = (base + rowbase) + lanes
            ok = (gvid >= ub) & (gvid < ub + C)
            pv = jnp.where(ok, pv, 0.0)
            return acc2 + pv

        return lax.fori_loop(0, C // L, group_body, acc)

    acc = lax.fori_loop(0, CHUNKS, chunk_body, jnp.zeros((L,), jnp.float32))
    out_v[...] = acc
    pltpu.sync_copy(out_v, out_hbm.at[wid])


_edge_loss_sc = pl.kernel(
    _edge_loss_body,
    out_type=jax.ShapeDtypeStruct((NW, L), jnp.float32),
    mesh=_mesh,
    compiler_params=pltpu.CompilerParams(needs_layout_passes=False),
    scratch_types=_SCRATCH,
)


def kernel(score, nidx, tidx, specweight):
    v, kn = nidx.shape
    km = kn - 1
    tflat = tidx[:, 0].astype(jnp.int32)
    swc = jnp.clip(specweight[:, 0], 0.0, 1.0)
    wfac = (1.0 - 0.9 * swc).astype(jnp.bfloat16)
    wbits = lax.bitcast_convert_type(wfac, jnp.uint16).astype(jnp.int32)
    packed = (tflat << 16) | wbits

    part = _edge_loss_sc(
        packed,
        nidx.reshape(v * kn),
        score.reshape(v * km),
        tflat,
        specweight[:, 0],
        )
    lossval = jnp.sum(part) / v
    return (score, lossval)


# speedup vs baseline: 117.1308x; 1.0016x over previous
import functools

import jax
import jax.numpy as jnp
from jax import lax
from jax.experimental import pallas as pl
from jax.experimental.pallas import tpu as pltpu
from jax.experimental.pallas import tpu_sc as plsc

NC = 2
NS = 16
NW = NC * NS
L = 16

V = 100000
KN = 33
KM = 32

VPT = 3136
C = 224
CHUNKS = 14

EPS = 1e-7
LN2 = 0.6931471805599453
P0 = -2.786812953867443
P1 = 5.046876044975941
P2 = -3.49249427987935
P3 = 1.5939013634991297
P4 = -0.4048671744191854
P5 = 0.043428907822139526

_mesh = plsc.VectorSubcoreMesh(core_axis_name="c", subcore_axis_name="s")

_SCRATCH = [
    pltpu.VMEM((V,), jnp.int32),
    pltpu.VMEM((C * KN,), jnp.int32),
    pltpu.VMEM((C * KM,), jnp.float32),
    pltpu.VMEM((C,), jnp.int32),
    pltpu.VMEM((C,), jnp.float32),
    pltpu.VMEM((C * KM,), jnp.float32),
    pltpu.VMEM((L,), jnp.float32),
]


def _edge_loss_body(packed_hbm, nidx_hbm, score_hbm, tidx_hbm, sw_hbm, out_hbm,
                    table_v, nidx_v, score_v, tidx_v, sw_v, term_v, out_v):
    wid = lax.axis_index("s") * NC + lax.axis_index("c")
    pltpu.sync_copy(packed_hbm, table_v)

    lanes = lax.iota(jnp.int32, L)
    lanes_km = lanes * KM

    def chunk_body(c, acc):
        ub = wid * VPT + c * C
        base = jnp.minimum(ub, V - C)
        pltpu.sync_copy(nidx_hbm.at[pl.ds(base * KN, C * KN)], nidx_v)
        pltpu.sync_copy(score_hbm.at[pl.ds(base * KM, C * KM)], score_v)
        pltpu.sync_copy(tidx_hbm.at[pl.ds(base, C)], tidx_v)
        pltpu.sync_copy(sw_hbm.at[pl.ds(base, C)], sw_v)

        @pl.loop(0, C, unroll=2)
        def _pass1(i):
            tv = plsc.load_gather(tidx_v, [jnp.full((L,), i, jnp.int32)])
            for h in range(2):
                nid = nidx_v[pl.ds(i * KN + 1 + h * L, L)]
                msk = nid >= 0
                safe = jnp.maximum(nid, 0)
                g = plsc.load_gather(table_v, [safe])
                nt = g >> 16
                wgt = plsc.bitcast(g << 16, jnp.float32)
                w = jnp.where(msk, wgt, 0.0)
                same = msk & (nt == tv)
                p = score_v[pl.ds(i * KM + h * L, L)]
                p = jnp.minimum(jnp.maximum(p, EPS), 1.0 - EPS)
                q = jnp.where(same, p, 1.0 - p)
                qb = plsc.bitcast(q, jnp.int32)
                e = (qb >> 23) - 127
                m = plsc.bitcast((qb & 0x007FFFFF) | 0x3F800000, jnp.float32)
                poly = P5
                for coef in (P4, P3, P2, P1, P0):
                    poly = poly * m + coef
                logq = (e.astype(jnp.float32) + poly) * LN2
                term_v[pl.ds(i * KM + h * L, L)] = (-logq) * w

        def group_body(vb, acc2):
            rowbase = vb * L
            num = jnp.zeros((L,), jnp.float32)
            den = jnp.zeros((L,), jnp.float32)
            for k in range(KM):
                idx = lanes_km + (rowbase * KM + k)
                t = plsc.load_gather(term_v, [idx])
                num = num + t
                den = den + jnp.where(t > 0.0, 1.0, 0.0)
            swv = sw_v[pl.ds(rowbase, L)]
            swv = jnp.minimum(jnp.maximum(swv, 0.0), 1.0)
            safe_den = jnp.where(den > 0.0, den, 1.0)
            pv = jnp.where(den > 0.0, num / safe_den, 0.0) * (1.0 - 0.9 * swv)
            gvid = (base + rowbase) + lanes
            ok = (gvid >= ub) & (gvid < ub + C)
            pv = jnp.where(ok, pv, 0.0)
            return acc2 + pv

        return lax.fori_loop(0, C // L, group_body, acc)

    acc = lax.fori_loop(0, CHUNKS, chunk_body, jnp.zeros((L,), jnp.float32))
    out_v[...] = acc
    pltpu.sync_copy(out_v, out_hbm.at[wid])


_edge_loss_sc = pl.kernel(
    _edge_loss_body,
    out_type=jax.ShapeDtypeStruct((NW, L), jnp.float32),
    mesh=_mesh,
    compiler_params=pltpu.CompilerParams(needs_layout_passes=False),
    scratch_types=_SCRATCH,
)


def kernel(score, nidx, tidx, specweight):
    v, kn = nidx.shape
    km = kn - 1
    tflat = tidx[:, 0].astype(jnp.int32)
    swc = jnp.clip(specweight[:, 0], 0.0, 1.0)
    wfac = (1.0 - 0.9 * swc).astype(jnp.bfloat16)
    wbits = lax.bitcast_convert_type(wfac, jnp.uint16).astype(jnp.int32)
    packed = (tflat << 16) | wbits
    tprobe = jnp.where(tflat < 0, -20, tflat)

    part = _edge_loss_sc(
        packed,
        nidx.reshape(v * kn),
        score.reshape(v * km),
        tprobe,
        specweight[:, 0],
        )
    lossval = jnp.sum(part) / v
    return (score, lossval)


# speedup vs baseline: 178.7869x; 1.5264x over previous
import functools

import jax
import jax.numpy as jnp
from jax import lax
from jax.experimental import pallas as pl
from jax.experimental.pallas import tpu as pltpu
from jax.experimental.pallas import tpu_sc as plsc

NC = 2
NS = 16
NW = NC * NS
L = 16

V = 100000
KN = 33
KM = 32

VPT = 3136
C = 224
CHUNKS = 14

EPS = 1e-7
LN2 = 0.6931471805599453
P0 = -2.786812953867443
P1 = 5.046876044975941
P2 = -3.49249427987935
P3 = 1.5939013634991297
P4 = -0.4048671744191854
P5 = 0.043428907822139526

_mesh = plsc.VectorSubcoreMesh(core_axis_name="c", subcore_axis_name="s")

_SCRATCH = [
    pltpu.VMEM((V,), jnp.int32),
    pltpu.VMEM((C * KN,), jnp.int32),
    pltpu.VMEM((C * KM,), jnp.float32),
    pltpu.VMEM((C,), jnp.int32),
    pltpu.VMEM((C,), jnp.float32),
    pltpu.VMEM((L,), jnp.float32),
]


def _edge_loss_body(packed_hbm, nidx_hbm, score_hbm, tidx_hbm, sw_hbm, out_hbm,
                    table_v, nidx_v, score_v, tidx_v, sw_v, out_v):
    wid = lax.axis_index("s") * NC + lax.axis_index("c")
    pltpu.sync_copy(packed_hbm, table_v)

    lanes = lax.iota(jnp.int32, L)
    lanes_km = lanes * KM
    lanes_kn = lanes * KN

    def chunk_body(c, acc):
        ub = wid * VPT + c * C
        base = jnp.minimum(ub, V - C)
        pltpu.sync_copy(nidx_hbm.at[pl.ds(base * KN, C * KN)], nidx_v)
        pltpu.sync_copy(score_hbm.at[pl.ds(base * KM, C * KM)], score_v)
        pltpu.sync_copy(tidx_hbm.at[pl.ds(base, C)], tidx_v)
        pltpu.sync_copy(sw_hbm.at[pl.ds(base, C)], sw_v)

        def group_body(vb, acc2):
            rowbase = vb * L
            tv = tidx_v[pl.ds(rowbase, L)]
            num0 = jnp.zeros((L,), jnp.float32)
            num1 = jnp.zeros((L,), jnp.float32)
            den0 = jnp.zeros((L,), jnp.float32)
            den1 = jnp.zeros((L,), jnp.float32)
            for k in range(KM):
                nid = plsc.load_gather(
                    nidx_v, [lanes_kn + (rowbase * KN + 1 + k)])
                msk = nid >= 0
                safe = jnp.maximum(nid, 0)
                g = plsc.load_gather(table_v, [safe])
                nt = g >> 16
                wgt = plsc.bitcast(g << 16, jnp.float32)
                w = jnp.where(msk, wgt, 0.0)
                wn = w * (-LN2)
                same = msk & (nt == tv)
                p = plsc.load_gather(
                    score_v, [lanes_km + (rowbase * KM + k)])
                q = jnp.where(same, p, 1.0 - p)
                qb = plsc.bitcast(q, jnp.int32)
                e = (qb >> 23) - 127
                m = plsc.bitcast((qb & 0x007FFFFF) | 0x3F800000, jnp.float32)
                poly = P5
                for coef in (P4, P3, P2, P1, P0):
                    poly = poly * m + coef
                term = (e.astype(jnp.float32) + poly) * wn
                if k % 2 == 0:
                    num0 = num0 + term
                    den0 = den0 + jnp.where(msk, 1.0, 0.0)
                else:
                    num1 = num1 + term
                    den1 = den1 + jnp.where(msk, 1.0, 0.0)
            num = num0 + num1
            den = den0 + den1
            swv = sw_v[pl.ds(rowbase, L)]
            swv = jnp.minimum(jnp.maximum(swv, 0.0), 1.0)
            safe_den = jnp.where(den > 0.0, den, 1.0)
            pv = jnp.where(den > 0.0, num / safe_den, 0.0) * (1.0 - 0.9 * swv)
            gvid = (base + rowbase) + lanes
            ok = (gvid >= ub) & (gvid < ub + C)
            pv = jnp.where(ok, pv, 0.0)
            return acc2 + pv

        return lax.fori_loop(0, C // L, group_body, acc)

    acc = lax.fori_loop(0, CHUNKS, chunk_body, jnp.zeros((L,), jnp.float32))
    out_v[...] = acc
    pltpu.sync_copy(out_v, out_hbm.at[wid])


_edge_loss_sc = pl.kernel(
    _edge_loss_body,
    out_type=jax.ShapeDtypeStruct((NW, L), jnp.float32),
    mesh=_mesh,
    compiler_params=pltpu.CompilerParams(needs_layout_passes=False),
    scratch_types=_SCRATCH,
)


def kernel(score, nidx, tidx, specweight):
    v, kn = nidx.shape
    km = kn - 1
    tflat = tidx[:, 0].astype(jnp.int32)
    swc = jnp.clip(specweight[:, 0], 0.0, 1.0)
    wfac = (1.0 - 0.9 * swc).astype(jnp.bfloat16)
    wbits = lax.bitcast_convert_type(wfac, jnp.uint16).astype(jnp.int32)
    packed = (tflat << 16) | wbits
    tprobe = jnp.where(tflat < 0, -20, tflat)

    part = _edge_loss_sc(
        packed,
        nidx.reshape(v * kn),
        score.reshape(v * km),
        tprobe,
        specweight[:, 0],
        )
    lossval = jnp.sum(part) / v
    return (score, lossval)


# speedup vs baseline: 203.9247x; 1.1406x over previous
import functools

import jax
import jax.numpy as jnp
from jax import lax
from jax.experimental import pallas as pl
from jax.experimental.pallas import tpu as pltpu
from jax.experimental.pallas import tpu_sc as plsc

NC = 2
NS = 16
NW = NC * NS
L = 16

V = 100000
KN = 33
KM = 32

VPT = 3136
C = 112
CHUNKS = 28

EPS = 1e-7
LN2 = 0.6931471805599453
P0 = -2.786812953867443
P1 = 5.046876044975941
P2 = -3.49249427987935
P3 = 1.5939013634991297
P4 = -0.4048671744191854
P5 = 0.043428907822139526

_mesh = plsc.VectorSubcoreMesh(core_axis_name="c", subcore_axis_name="s")

_SCRATCH = [
    pltpu.VMEM((V,), jnp.int32),
    pltpu.VMEM((2 * C * KN,), jnp.int32),
    pltpu.VMEM((2 * C * KM,), jnp.float32),
    pltpu.VMEM((2 * C,), jnp.int32),
    pltpu.VMEM((2 * C,), jnp.float32),
    pltpu.VMEM((L,), jnp.float32),
    pltpu.SemaphoreType.DMA,
    pltpu.SemaphoreType.DMA,
]


def _edge_loss_body(packed_hbm, nidx_hbm, score_hbm, tidx_hbm, sw_hbm, out_hbm,
                    table_v, nidx_v, score_v, tidx_v, sw_v, out_v,
                    sem0, sem1):
    wid = lax.axis_index("s") * NC + lax.axis_index("c")
    pltpu.sync_copy(packed_hbm, table_v)

    lanes = lax.iota(jnp.int32, L)
    lanes_km = lanes * KM
    lanes_kn = lanes * KN

    def slot_refs(s):
        return (nidx_v.at[pl.ds(s * C * KN, C * KN)],
                score_v.at[pl.ds(s * C * KM, C * KM)],
                tidx_v.at[pl.ds(s * C, C)],
                sw_v.at[pl.ds(s * C, C)])

    def hbm_slices(c):
        ub = wid * VPT + c * C
        base = jnp.minimum(ub, V - C)
        return (nidx_hbm.at[pl.ds(base * KN, C * KN)],
                score_hbm.at[pl.ds(base * KM, C * KM)],
                tidx_hbm.at[pl.ds(base, C)],
                sw_hbm.at[pl.ds(base, C)])

    def fire(s, sem, c):
        for src, dst in zip(hbm_slices(c), slot_refs(s)):
            pltpu.async_copy(src, dst, sem)

    def drain(s, sem):
        for src, dst in zip(hbm_slices(0), slot_refs(s)):
            pltpu.make_async_copy(src, dst, sem).wait()

    def compute(s, c, acc):
        ub = wid * VPT + c * C
        base = jnp.minimum(ub, V - C)
        nb, sb, tb, wb = s * C * KN, s * C * KM, s * C, s * C

        def group_body(vb, acc2):
            rowbase = vb * L
            tv = tidx_v[pl.ds(tb + rowbase, L)]
            num0 = jnp.zeros((L,), jnp.float32)
            num1 = jnp.zeros((L,), jnp.float32)
            den0 = jnp.zeros((L,), jnp.float32)
            den1 = jnp.zeros((L,), jnp.float32)
            for k in range(KM):
                nid = plsc.load_gather(
                    nidx_v, [lanes_kn + (nb + rowbase * KN + 1 + k)])
                msk = nid >= 0
                safe = jnp.maximum(nid, 0)
                g = plsc.load_gather(table_v, [safe])
                nt = g >> 16
                wgt = plsc.bitcast(g << 16, jnp.float32)
                w = jnp.where(msk, wgt, 0.0)
                wn = w * (-LN2)
                same = msk & (nt == tv)
                p = plsc.load_gather(
                    score_v, [lanes_km + (sb + rowbase * KM + k)])
                q = jnp.where(same, p, 1.0 - p)
                qb = plsc.bitcast(q, jnp.int32)
                e = (qb >> 23) - 127
                m = plsc.bitcast((qb & 0x007FFFFF) | 0x3F800000, jnp.float32)
                poly = P5
                for coef in (P4, P3, P2, P1, P0):
                    poly = poly * m + coef
                term = (e.astype(jnp.float32) + poly) * wn
                if k % 2 == 0:
                    num0 = num0 + term
                    den0 = den0 + jnp.where(msk, 1.0, 0.0)
                else:
                    num1 = num1 + term
                    den1 = den1 + jnp.where(msk, 1.0, 0.0)
            num = num0 + num1
            den = den0 + den1
            swv = sw_v[pl.ds(wb + rowbase, L)]
            swv = jnp.minimum(jnp.maximum(swv, 0.0), 1.0)
            safe_den = jnp.where(den > 0.0, den, 1.0)
            pv = jnp.where(den > 0.0, num / safe_den, 0.0) * (1.0 - 0.9 * swv)
            gvid = (base + rowbase) + lanes
            ok = (gvid >= ub) & (gvid < ub + C)
            pv = jnp.where(ok, pv, 0.0)
            return acc2 + pv

        return lax.fori_loop(0, C // L, group_body, acc)

    fire(0, sem0, 0)
    fire(1, sem1, 1)

    def chunk_pair(c2, acc):
        c0 = 2 * c2
        drain(0, sem0)
        acc = compute(0, c0, acc)
        fire(0, sem0, c0 + 2)
        drain(1, sem1)
        acc = compute(1, c0 + 1, acc)
        fire(1, sem1, c0 + 3)
        return acc

    acc = lax.fori_loop(0, CHUNKS // 2, chunk_pair,
                        jnp.zeros((L,), jnp.float32))
    drain(0, sem0)
    drain(1, sem1)
    out_v[...] = acc
    pltpu.sync_copy(out_v, out_hbm.at[wid])


_edge_loss_sc = pl.kernel(
    _edge_loss_body,
    out_type=jax.ShapeDtypeStruct((NW, L), jnp.float32),
    mesh=_mesh,
    compiler_params=pltpu.CompilerParams(needs_layout_passes=False),
    scratch_types=_SCRATCH,
)


def kernel(score, nidx, tidx, specweight):
    v, kn = nidx.shape
    km = kn - 1
    tflat = tidx[:, 0].astype(jnp.int32)
    swc = jnp.clip(specweight[:, 0], 0.0, 1.0)
    wfac = (1.0 - 0.9 * swc).astype(jnp.bfloat16)
    wbits = lax.bitcast_convert_type(wfac, jnp.uint16).astype(jnp.int32)
    packed = (tflat << 16) | wbits
    tprobe = jnp.where(tflat < 0, -20, tflat)

    part = _edge_loss_sc(
        packed,
        nidx.reshape(v * kn),
        score.reshape(v * km),
        tprobe,
        specweight[:, 0],
        )
    lossval = jnp.sum(part) / v
    return (score, lossval)


# speedup vs baseline: 252.0299x; 1.2359x over previous
import functools

import jax
import jax.numpy as jnp
from jax import lax
from jax.experimental import pallas as pl
from jax.experimental.pallas import tpu as pltpu
from jax.experimental.pallas import tpu_sc as plsc

NC = 2
NS = 16
NW = NC * NS
L = 16

V = 100000
KN = 33
KM = 32

VPT = 3328
C = 128
CHUNKS = 26
NROW = 40
BT_MAX = 99968
BS_MAX = V - C

EPS = 1e-7
LN2 = 0.6931471805599453
P0 = -2.786812953867443
P1 = 5.046876044975941
P2 = -3.49249427987935
P3 = 1.5939013634991297
P4 = -0.4048671744191854
P5 = 0.043428907822139526

_mesh = plsc.VectorSubcoreMesh(core_axis_name="c", subcore_axis_name="s")

_ROWBLOCKS = ((0, 8), (8, 8), (16, 8), (24, 8), (32, 1))

_SCRATCH = [
    pltpu.VMEM((V,), jnp.int32),
    pltpu.VMEM((2 * NROW, C), jnp.int32),
    pltpu.VMEM((2 * C * KM,), jnp.float32),
    pltpu.VMEM((2 * C,), jnp.int32),
    pltpu.VMEM((2 * C,), jnp.float32),
    pltpu.VMEM((L,), jnp.float32),
    pltpu.SemaphoreType.DMA,
    pltpu.SemaphoreType.DMA,
]


def _edge_loss_body(packed_hbm, nidxt_hbm, score_hbm, tidx_hbm, sw_hbm,
                    out_hbm, table_v, nidx_v, score_v, tidx_v, sw_v, out_v,
                    sem0, sem1):
    wid = lax.axis_index("s") * NC + lax.axis_index("c")
    pltpu.sync_copy(packed_hbm, table_v)

    lanes = lax.iota(jnp.int32, L)
    lanes_km = lanes * KM

    def copies(s, c):
        ub = wid * VPT + c * C
        bt = jnp.minimum(ub, BT_MAX)
        bs = jnp.minimum(ub, BS_MAX)
        pairs = [(nidxt_hbm.at[pl.ds(rb, nr), pl.ds(bt, C)],
                  nidx_v.at[pl.ds(s * NROW + rb, nr), :])
                 for rb, nr in _ROWBLOCKS]
        pairs += [
            (score_hbm.at[pl.ds(bs * KM, C * KM)],
             score_v.at[pl.ds(s * C * KM, C * KM)]),
            (tidx_hbm.at[pl.ds(bs, C)], tidx_v.at[pl.ds(s * C, C)]),
            (sw_hbm.at[pl.ds(bs, C)], sw_v.at[pl.ds(s * C, C)]),
        ]
        return pairs

    def fire(s, sem, c):
        for src, dst in copies(s, c):
            pltpu.async_copy(src, dst, sem)

    def drain(s, sem):
        for src, dst in copies(s, 0):
            pltpu.make_async_copy(src, dst, sem).wait()

    def compute(s, c, acc):
        ub = wid * VPT + c * C
        bt = jnp.minimum(ub, BT_MAX)
        bs = jnp.minimum(ub, BS_MAX)
        d = bt - bs

        def group_body(vb, acc2):
            rowbase = vb * L
            tv = tidx_v[pl.ds(s * C + d + rowbase, L)]
            soff = (s * C + d + rowbase) * KM
            num0 = jnp.zeros((L,), jnp.float32)
            num1 = jnp.zeros((L,), jnp.float32)
            den0 = jnp.zeros((L,), jnp.float32)
            den1 = jnp.zeros((L,), jnp.float32)
            for k in range(1, KN):
                row = s * NROW + (k if k < 32 else 32)
                nid = nidx_v[row, pl.ds(rowbase, L)]
                msk = nid >= 0
                safe = jnp.minimum(jnp.maximum(nid, 0), V - 1)
                g = plsc.load_gather(table_v, [safe])
                nt = g >> 16
                wgt = plsc.bitcast(g << 16, jnp.float32)
                w = jnp.where(msk, wgt, 0.0)
                wn = w * (-LN2)
                same = msk & (nt == tv)
                p = plsc.load_gather(score_v, [lanes_km + (soff + (k - 1))])
                q = jnp.where(same, p, 1.0 - p)
                qb = plsc.bitcast(q, jnp.int32)
                e = (qb >> 23) - 127
                m = plsc.bitcast((qb & 0x007FFFFF) | 0x3F800000, jnp.float32)
                poly = P5
                for coef in (P4, P3, P2, P1, P0):
                    poly = poly * m + coef
                term = (e.astype(jnp.float32) + poly) * wn
                if k % 2 == 0:
                    num0 = num0 + term
                    den0 = den0 + jnp.where(msk, 1.0, 0.0)
                else:
                    num1 = num1 + term
                    den1 = den1 + jnp.where(msk, 1.0, 0.0)
            num = num0 + num1
            den = den0 + den1
            swv = sw_v[pl.ds(s * C + d + rowbase, L)]
            swv = jnp.minimum(jnp.maximum(swv, 0.0), 1.0)
            safe_den = jnp.where(den > 0.0, den, 1.0)
            pv = jnp.where(den > 0.0, num / safe_den, 0.0) * (1.0 - 0.9 * swv)
            gvid = (bt + rowbase) + lanes
            ok = (gvid >= ub) & (gvid < ub + C) & (gvid < V)
            pv = jnp.where(ok, pv, 0.0)
            return acc2 + pv

        return lax.fori_loop(0, C // L, group_body, acc)

    fire(0, sem0, 0)
    fire(1, sem1, 1)

    def chunk_pair(c2, acc):
        c0 = 2 * c2
        drain(0, sem0)
        acc = compute(0, c0, acc)
        fire(0, sem0, c0 + 2)
        drain(1, sem1)
        acc = compute(1, c0 + 1, acc)
        fire(1, sem1, c0 + 3)
        return acc

    acc = lax.fori_loop(0, CHUNKS // 2, chunk_pair,
                        jnp.zeros((L,), jnp.float32))
    drain(0, sem0)
    drain(1, sem1)
    out_v[...] = acc
    pltpu.sync_copy(out_v, out_hbm.at[wid])


_edge_loss_sc = pl.kernel(
    _edge_loss_body,
    out_type=jax.ShapeDtypeStruct((NW, L), jnp.float32),
    mesh=_mesh,
    compiler_params=pltpu.CompilerParams(needs_layout_passes=False),
    scratch_types=_SCRATCH,
)


def kernel(score, nidx, tidx, specweight):
    v, kn = nidx.shape
    km = kn - 1
    tflat = tidx[:, 0].astype(jnp.int32)
    swc = jnp.clip(specweight[:, 0], 0.0, 1.0)
    wfac = (1.0 - 0.9 * swc).astype(jnp.bfloat16)
    wbits = lax.bitcast_convert_type(wfac, jnp.uint16).astype(jnp.int32)
    packed = (tflat << 16) | wbits
    tprobe = jnp.where(tflat < 0, -20, tflat)

    part = _edge_loss_sc(
        packed,
        nidx.T,
        score.reshape(v * km),
        tprobe,
        specweight[:, 0],
        )
    lossval = jnp.sum(part) / v
    return (score, lossval)


# speedup vs baseline: 428.2488x; 1.6992x over previous
import functools

import jax
import jax.numpy as jnp
from jax import lax
from jax.experimental import pallas as pl
from jax.experimental.pallas import tpu as pltpu
from jax.experimental.pallas import tpu_sc as plsc

NC = 2
NS = 16
NW = NC * NS
L = 16

V = 100000
KN = 33
KM = 32

VPT = 3328
C = 128
CHUNKS = 26
NROW = 40
BT_MAX = 99968

EPS = 1e-7
LN2 = 0.6931471805599453
P0 = -2.786812953867443
P1 = 5.046876044975941
P2 = -3.49249427987935
P3 = 1.5939013634991297
P4 = -0.4048671744191854
P5 = 0.043428907822139526

_mesh = plsc.VectorSubcoreMesh(core_axis_name="c", subcore_axis_name="s")

_ROWBLOCKS = ((0, 8), (8, 8), (16, 8), (24, 8), (32, 1))
_SROWBLOCKS = ((0, 8), (8, 8), (16, 8), (24, 8))

_SCRATCH = [
    pltpu.VMEM((V,), jnp.int32),
    pltpu.VMEM((2 * NROW, C), jnp.int32),
    pltpu.VMEM((2 * KM, C), jnp.float32),
    pltpu.VMEM((2 * C,), jnp.int32),
    pltpu.VMEM((2 * C,), jnp.float32),
    pltpu.VMEM((L,), jnp.float32),
    pltpu.SemaphoreType.DMA,
    pltpu.SemaphoreType.DMA,
]


def _edge_loss_body(packed_hbm, nidxt_hbm, scoret_hbm, tidx_hbm, sw_hbm,
                    out_hbm, table_v, nidx_v, score_v, tidx_v, sw_v, out_v,
                    sem0, sem1):
    wid = lax.axis_index("s") * NC + lax.axis_index("c")
    pltpu.sync_copy(packed_hbm, table_v)

    lanes = lax.iota(jnp.int32, L)

    def copies(s, c):
        ub = wid * VPT + c * C
        bt = jnp.minimum(ub, BT_MAX)
        pairs = [(nidxt_hbm.at[pl.ds(rb, nr), pl.ds(bt, C)],
                  nidx_v.at[pl.ds(s * NROW + rb, nr), :])
                 for rb, nr in _ROWBLOCKS]
        pairs += [(scoret_hbm.at[pl.ds(rb, nr), pl.ds(bt, C)],
                   score_v.at[pl.ds(s * KM + rb, nr), :])
                  for rb, nr in _SROWBLOCKS]
        pairs += [
            (tidx_hbm.at[pl.ds(bt, C)], tidx_v.at[pl.ds(s * C, C)]),
            (sw_hbm.at[pl.ds(bt, C)], sw_v.at[pl.ds(s * C, C)]),
        ]
        return pairs

    def fire(s, sem, c):
        for src, dst in copies(s, c):
            pltpu.async_copy(src, dst, sem)

    def drain(s, sem):
        for src, dst in copies(s, 0):
            pltpu.make_async_copy(src, dst, sem).wait()

    def compute(s, c, acc):
        ub = wid * VPT + c * C
        bt = jnp.minimum(ub, BT_MAX)

        def group_body(vb, acc2):
            rowbase = vb * L
            tv = tidx_v[pl.ds(s * C + rowbase, L)]
            num0 = jnp.zeros((L,), jnp.float32)
            num1 = jnp.zeros((L,), jnp.float32)
            den0 = jnp.zeros((L,), jnp.float32)
            den1 = jnp.zeros((L,), jnp.float32)
            for k in range(1, KN):
                nid = nidx_v[s * NROW + k, pl.ds(rowbase, L)]
                msk = nid >= 0
                safe = jnp.minimum(jnp.maximum(nid, 0), V - 1)
                g = plsc.load_gather(table_v, [safe])
                nt = g >> 16
                wgt = plsc.bitcast(g << 16, jnp.float32)
                w = jnp.where(msk, wgt, 0.0)
                wn = w * (-LN2)
                same = msk & (nt == tv)
                p = score_v[s * KM + (k - 1), pl.ds(rowbase, L)]
                q = jnp.where(same, p, 1.0 - p)
                qb = plsc.bitcast(q, jnp.int32)
                e = (qb >> 23) - 127
                m = plsc.bitcast((qb & 0x007FFFFF) | 0x3F800000, jnp.float32)
                poly = P5
                for coef in (P4, P3, P2, P1, P0):
                    poly = poly * m + coef
                term = (e.astype(jnp.float32) + poly) * wn
                if k % 2 == 0:
                    num0 = num0 + term
                    den0 = den0 + jnp.where(msk, 1.0, 0.0)
                else:
                    num1 = num1 + term
                    den1 = den1 + jnp.where(msk, 1.0, 0.0)
            num = num0 + num1
            den = den0 + den1
            swv = sw_v[pl.ds(s * C + rowbase, L)]
            swv = jnp.minimum(jnp.maximum(swv, 0.0), 1.0)
            safe_den = jnp.where(den > 0.0, den, 1.0)
            pv = jnp.where(den > 0.0, num / safe_den, 0.0) * (1.0 - 0.9 * swv)
            gvid = (bt + rowbase) + lanes
            ok = (gvid >= ub) & (gvid < ub + C) & (gvid < V)
            pv = jnp.where(ok, pv, 0.0)
            return acc2 + pv

        return lax.fori_loop(0, C // L, group_body, acc)

    fire(0, sem0, 0)
    fire(1, sem1, 1)

    def chunk_pair(c2, acc):
        c0 = 2 * c2
        drain(0, sem0)
        acc = compute(0, c0, acc)
        fire(0, sem0, c0 + 2)
        drain(1, sem1)
        acc = compute(1, c0 + 1, acc)
        fire(1, sem1, c0 + 3)
        return acc

    acc = lax.fori_loop(0, CHUNKS // 2, chunk_pair,
                        jnp.zeros((L,), jnp.float32))
    drain(0, sem0)
    drain(1, sem1)
    out_v[...] = acc
    pltpu.sync_copy(out_v, out_hbm.at[wid])


_edge_loss_sc = pl.kernel(
    _edge_loss_body,
    out_type=jax.ShapeDtypeStruct((NW, L), jnp.float32),
    mesh=_mesh,
    compiler_params=pltpu.CompilerParams(needs_layout_passes=False),
    scratch_types=_SCRATCH,
)


def kernel(score, nidx, tidx, specweight):
    v, kn = nidx.shape
    km = kn - 1
    tflat = tidx[:, 0].astype(jnp.int32)
    swc = jnp.clip(specweight[:, 0], 0.0, 1.0)
    wfac = (1.0 - 0.9 * swc).astype(jnp.bfloat16)
    wbits = lax.bitcast_convert_type(wfac, jnp.uint16).astype(jnp.int32)
    packed = (tflat << 16) | wbits
    tprobe = jnp.where(tflat < 0, -20, tflat)

    part = _edge_loss_sc(
        packed,
        nidx.T,
        score[:, :, 0].T,
        tprobe,
        specweight[:, 0],
        )
    lossval = jnp.sum(part) / v
    return (score, lossval)


# speedup vs baseline: 486.6979x; 1.1365x over previous
import functools

import jax
import jax.numpy as jnp
from jax import lax
from jax.experimental import pallas as pl
from jax.experimental.pallas import tpu as pltpu
from jax.experimental.pallas import tpu_sc as plsc

NC = 2
NS = 16
NW = NC * NS
L = 16

V = 100000
KN = 33
KM = 32

VPT = 3328
C = 128
CHUNKS = 26
NROW = 40
BT_MAX = 99968

EPS = 1e-7
LN2 = 0.6931471805599453
P0 = -2.786812953867443
P1 = 5.046876044975941
P2 = -3.49249427987935
P3 = 1.5939013634991297
P4 = -0.4048671744191854
P5 = 0.043428907822139526

_mesh = plsc.VectorSubcoreMesh(core_axis_name="c", subcore_axis_name="s")

_ROWBLOCKS = ((0, 8), (8, 8), (16, 8), (24, 8), (32, 1))
_SROWBLOCKS = ((0, 8), (8, 8), (16, 8), (24, 8))

_SCRATCH = [
    pltpu.VMEM((V,), jnp.int32),
    pltpu.VMEM((2 * NROW, C), jnp.int32),
    pltpu.VMEM((2 * KM, 1, C), jnp.float32),
    pltpu.VMEM((2 * C,), jnp.int32),
    pltpu.VMEM((2 * C,), jnp.float32),
    pltpu.VMEM((L,), jnp.float32),
    pltpu.SemaphoreType.DMA,
    pltpu.SemaphoreType.DMA,
]


def _edge_loss_body(packed_hbm, nidxt_hbm, scoret_hbm, tidx_hbm, sw_hbm,
                    out_hbm, table_v, nidx_v, score_v, tidx_v, sw_v, out_v,
                    sem0, sem1):
    wid = lax.axis_index("s") * NC + lax.axis_index("c")
    pltpu.sync_copy(packed_hbm, table_v)

    lanes = lax.iota(jnp.int32, L)

    def copies(s, c):
        ub = wid * VPT + c * C
        bt = jnp.minimum(ub, BT_MAX)
        pairs = [(nidxt_hbm.at[pl.ds(rb, nr), pl.ds(bt, C)],
                  nidx_v.at[pl.ds(s * NROW + rb, nr), :])
                 for rb, nr in _ROWBLOCKS]
        pairs += [(scoret_hbm.at[pl.ds(rb, nr), :, pl.ds(bt, C)],
                   score_v.at[pl.ds(s * KM + rb, nr), :, :])
                  for rb, nr in _SROWBLOCKS]
        pairs += [
            (tidx_hbm.at[pl.ds(bt, C)], tidx_v.at[pl.ds(s * C, C)]),
            (sw_hbm.at[pl.ds(bt, C)], sw_v.at[pl.ds(s * C, C)]),
        ]
        return pairs

    def fire(s, sem, c):
        for src, dst in copies(s, c):
            pltpu.async_copy(src, dst, sem)

    def drain(s, sem):
        for src, dst in copies(s, 0):
            pltpu.make_async_copy(src, dst, sem).wait()

    def compute(s, c, acc):
        ub = wid * VPT + c * C
        bt = jnp.minimum(ub, BT_MAX)

        def group_body(vb, acc2):
            rowbase = vb * L
            tv = tidx_v[pl.ds(s * C + rowbase, L)]
            num0 = jnp.zeros((L,), jnp.float32)
            num1 = jnp.zeros((L,), jnp.float32)
            den0 = jnp.zeros((L,), jnp.float32)
            den1 = jnp.zeros((L,), jnp.float32)
            for k in range(1, KN):
                nid = nidx_v[s * NROW + k, pl.ds(rowbase, L)]
                msk = nid >= 0
                safe = jnp.minimum(jnp.maximum(nid, 0), V - 1)
                g = plsc.load_gather(table_v, [safe])
                nt = g >> 16
                wgt = plsc.bitcast(g << 16, jnp.float32)
                w = jnp.where(msk, wgt, 0.0)
                wn = w * (-LN2)
                same = msk & (nt == tv)
                p = score_v[s * KM + (k - 1), 0, pl.ds(rowbase, L)]
                q = jnp.where(same, p, 1.0 - p)
                qb = plsc.bitcast(q, jnp.int32)
                e = (qb >> 23) - 127
                m = plsc.bitcast((qb & 0x007FFFFF) | 0x3F800000, jnp.float32)
                poly = P5
                for coef in (P4, P3, P2, P1, P0):
                    poly = poly * m + coef
                term = (e.astype(jnp.float32) + poly) * wn
                if k % 2 == 0:
                    num0 = num0 + term
                    den0 = den0 + jnp.where(msk, 1.0, 0.0)
                else:
                    num1 = num1 + term
                    den1 = den1 + jnp.where(msk, 1.0, 0.0)
            num = num0 + num1
            den = den0 + den1
            swv = sw_v[pl.ds(s * C + rowbase, L)]
            swv = jnp.minimum(jnp.maximum(swv, 0.0), 1.0)
            safe_den = jnp.where(den > 0.0, den, 1.0)
            pv = jnp.where(den > 0.0, num / safe_den, 0.0) * (1.0 - 0.9 * swv)
            gvid = (bt + rowbase) + lanes
            ok = (gvid >= ub) & (gvid < ub + C) & (gvid < V)
            pv = jnp.where(ok, pv, 0.0)
            return acc2 + pv

        return lax.fori_loop(0, C // L, group_body, acc)

    fire(0, sem0, 0)
    fire(1, sem1, 1)

    def chunk_pair(c2, acc):
        c0 = 2 * c2
        drain(0, sem0)
        acc = compute(0, c0, acc)
        fire(0, sem0, c0 + 2)
        drain(1, sem1)
        acc = compute(1, c0 + 1, acc)
        fire(1, sem1, c0 + 3)
        return acc

    acc = lax.fori_loop(0, CHUNKS // 2, chunk_pair,
                        jnp.zeros((L,), jnp.float32))
    drain(0, sem0)
    drain(1, sem1)
    out_v[...] = acc
    pltpu.sync_copy(out_v, out_hbm.at[wid])


_edge_loss_sc = pl.kernel(
    _edge_loss_body,
    out_type=jax.ShapeDtypeStruct((NW, L), jnp.float32),
    mesh=_mesh,
    compiler_params=pltpu.CompilerParams(needs_layout_passes=False),
    scratch_types=_SCRATCH,
)


def kernel(score, nidx, tidx, specweight):
    v, kn = nidx.shape
    km = kn - 1
    tflat = tidx[:, 0].astype(jnp.int32)
    swc = jnp.clip(specweight[:, 0], 0.0, 1.0)
    wfac = (1.0 - 0.9 * swc).astype(jnp.bfloat16)
    wbits = lax.bitcast_convert_type(wfac, jnp.uint16).astype(jnp.int32)
    packed = (tflat << 16) | wbits
    tprobe = jnp.where(tflat < 0, -20, tflat)

    part = _edge_loss_sc(
        packed,
        nidx.T,
        jnp.transpose(score, (1, 2, 0)),
        tprobe,
        specweight[:, 0],
        )
    lossval = jnp.sum(part) / v
    return (score, lossval)


# speedup vs baseline: 505.2644x; 1.0381x over previous
import jax
import jax.numpy as jnp
from jax import lax
from jax.experimental import pallas as pl
from jax.experimental.pallas import tpu as pltpu
from jax.experimental.pallas import tpu_sc as plsc

NC = 2
NS = 16
NW = NC * NS
L = 16

V = 100000
KN = 33
KM = 32

VPT = 3328
C = 128
CHUNKS = 26
NROW = 40
BT_MAX = 99968

LN2 = 0.6931471805599453
P0 = -2.496773767905599
P1 = 4.0283727668469735
P2 = -2.0810602034595114
P3 = 0.6288157291849531
P4 = -0.07915036575317282

_mesh = plsc.VectorSubcoreMesh(core_axis_name="c", subcore_axis_name="s")

_ROWBLOCKS = ((0, 8), (8, 8), (16, 8), (24, 8), (32, 1))
_SROWBLOCKS = ((0, 8), (8, 8), (16, 8), (24, 8))

_SCRATCH = [
    pltpu.VMEM((V,), jnp.int32),
    pltpu.VMEM((2 * NROW, C), jnp.int32),
    pltpu.VMEM((2 * KM, 1, C), jnp.float32),
    pltpu.VMEM((2 * C,), jnp.int32),
    pltpu.VMEM((2 * C,), jnp.float32),
    pltpu.VMEM((L,), jnp.float32),
    pltpu.SemaphoreType.DMA,
    pltpu.SemaphoreType.DMA,
]


def _edge_loss_body(packed_hbm, nidxt_hbm, scoret_hbm, tidx_hbm, sw_hbm,
                    out_hbm, table_v, nidx_v, score_v, tidx_v, sw_v, out_v,
                    sem0, sem1):
    wid = lax.axis_index("s") * NC + lax.axis_index("c")
    pltpu.sync_copy(packed_hbm, table_v)

    lanes = lax.iota(jnp.int32, L)

    def copies(s, c):
        ub = wid * VPT + c * C
        bt = jnp.minimum(ub, BT_MAX)
        pairs = [(nidxt_hbm.at[pl.ds(rb, nr), pl.ds(bt, C)],
                  nidx_v.at[pl.ds(s * NROW + rb, nr), :])
                 for rb, nr in _ROWBLOCKS]
        pairs += [(scoret_hbm.at[pl.ds(rb, nr), :, pl.ds(bt, C)],
                   score_v.at[pl.ds(s * KM + rb, nr), :, :])
                  for rb, nr in _SROWBLOCKS]
        pairs += [
            (tidx_hbm.at[pl.ds(bt, C)], tidx_v.at[pl.ds(s * C, C)]),
            (sw_hbm.at[pl.ds(bt, C)], sw_v.at[pl.ds(s * C, C)]),
        ]
        return pairs

    def fire(s, sem, c):
        for src, dst in copies(s, c):
            pltpu.async_copy(src, dst, sem)

    def drain(s, sem):
        for src, dst in copies(s, 0):
            pltpu.make_async_copy(src, dst, sem).wait()

    def compute(s, c, acc):
        ub = wid * VPT + c * C
        bt = jnp.minimum(ub, BT_MAX)

        def group_body(vb, acc2):
            rowbase = vb * L
            tv = tidx_v[pl.ds(s * C + rowbase, L)]
            num0 = jnp.zeros((L,), jnp.float32)
            num1 = jnp.zeros((L,), jnp.float32)
            den0 = jnp.zeros((L,), jnp.float32)
            den1 = jnp.zeros((L,), jnp.float32)
            for k in range(1, KN):
                nid = nidx_v[s * NROW + k, pl.ds(rowbase, L)]
                msk = nid >= 0
                safe = jnp.minimum(jnp.maximum(nid, 0), V - 1)
                g = plsc.load_gather(table_v, [safe])
                nt = g >> 16
                wgt = plsc.bitcast(g << 16, jnp.float32)
                w = jnp.where(msk, wgt, 0.0)
                wn = w * (-LN2)
                same = msk & (nt == tv)
                p = score_v[s * KM + (k - 1), 0, pl.ds(rowbase, L)]
                q = jnp.where(same, p, 1.0 - p)
                qb = plsc.bitcast(q, jnp.int32)
                e = (qb >> 23) - 127
                m = plsc.bitcast((qb & 0x007FFFFF) | 0x3F800000, jnp.float32)
                poly = P4
                for coef in (P3, P2, P1, P0):
                    poly = poly * m + coef
                term = (e.astype(jnp.float32) + poly) * wn
                if k % 2 == 0:
                    num0 = num0 + term
                    den0 = den0 + jnp.where(msk, 1.0, 0.0)
                else:
                    num1 = num1 + term
                    den1 = den1 + jnp.where(msk, 1.0, 0.0)
            num = num0 + num1
            den = den0 + den1
            swv = sw_v[pl.ds(s * C + rowbase, L)]
            swv = jnp.minimum(jnp.maximum(swv, 0.0), 1.0)
            safe_den = jnp.where(den > 0.0, den, 1.0)
            pv = jnp.where(den > 0.0, num / safe_den, 0.0) * (1.0 - 0.9 * swv)
            gvid = (bt + rowbase) + lanes
            ok = (gvid >= ub) & (gvid < ub + C) & (gvid < V)
            pv = jnp.where(ok, pv, 0.0)
            return acc2 + pv

        return lax.fori_loop(0, C // L, group_body, acc)

    fire(0, sem0, 0)
    fire(1, sem1, 1)

    def chunk_pair(c2, acc):
        c0 = 2 * c2
        drain(0, sem0)
        acc = compute(0, c0, acc)
        fire(0, sem0, c0 + 2)
        drain(1, sem1)
        acc = compute(1, c0 + 1, acc)
        fire(1, sem1, c0 + 3)
        return acc

    acc = lax.fori_loop(0, CHUNKS // 2, chunk_pair,
                        jnp.zeros((L,), jnp.float32))
    drain(0, sem0)
    drain(1, sem1)
    out_v[...] = acc
    pltpu.sync_copy(out_v, out_hbm.at[wid])


_edge_loss_sc = pl.kernel(
    _edge_loss_body,
    out_type=jax.ShapeDtypeStruct((NW, L), jnp.float32),
    mesh=_mesh,
    compiler_params=pltpu.CompilerParams(needs_layout_passes=False),
    scratch_types=_SCRATCH,
)


def kernel(score, nidx, tidx, specweight):
    v, kn = nidx.shape
    km = kn - 1
    tflat = tidx[:, 0].astype(jnp.int32)
    swc = jnp.clip(specweight[:, 0], 0.0, 1.0)
    wfac = (1.0 - 0.9 * swc).astype(jnp.bfloat16)
    wbits = lax.bitcast_convert_type(wfac, jnp.uint16).astype(jnp.int32)
    packed = (tflat << 16) | wbits
    tprobe = jnp.where(tflat < 0, -20, tflat)

    part = _edge_loss_sc(
        packed,
        nidx.T,
        jnp.transpose(score, (1, 2, 0)),
        tprobe,
        specweight[:, 0],
        )
    lossval = jnp.sum(part) / v
    return (score, lossval)
